# baseline (speedup 1.0000x reference)
import functools

import jax
import jax.numpy as jnp
from jax.experimental import pallas as pl

B, V = 128, 100000
K = 10
BR = 8


def _cumsum_cols(a):
    acc = a[:, 0:1]
    cols = [acc]
    for r in range(1, a.shape[1]):
        acc = acc + a[:, r:r + 1]
        cols.append(acc)
    return jnp.concatenate(cols, axis=1)


def _body(x_ref, t_ref, q_ref, mk_ref, u_ref, o_ref):
    x = x_ref[...]
    T = t_ref[0, 0]
    Q = q_ref[0, 0]
    mk = mk_ref[...]
    u = u_ref[...]

    col = jax.lax.broadcasted_iota(jnp.int32, x.shape, 1)
    neg_inf = jnp.float32(-jnp.inf)

    work = x
    vals = []
    for r in range(K):
        m = jnp.max(work, axis=1, keepdims=True)
        vals.append(m)
        if r + 1 < K:
            first = jnp.argmax(work, axis=1).astype(jnp.int32)[:, None]
            work = jnp.where(col == first, neg_inf, work)

    vs = jnp.concatenate(vals, axis=1)

    M = vs[:, 0:1]
    tope = jnp.exp((vs - M) / T)
    Z = jnp.sum(jnp.exp((x - M) / T), axis=1, keepdims=True)
    p = tope / Z

    ordered = p + mk
    cums = _cumsum_cols(p) + _cumsum_cols(mk)

    sizes_base = 1 + jnp.sum((cums <= Q).astype(jnp.int32), axis=1,
                             keepdims=True)
    rk = jax.lax.broadcasted_iota(jnp.int32, (BR, K), 1)
    sel = rk == (sizes_base - 1)
    ord_at = jnp.sum(jnp.where(sel, ordered, 0.0), axis=1, keepdims=True)
    cum_at = jnp.sum(jnp.where(sel, cums, 0.0), axis=1, keepdims=True)
    vprob = (Q - (cum_at - ord_at)) / ord_at
    sizes = sizes_base - (u >= vprob).astype(jnp.int32)

    sel2 = rk == (sizes - 1)
    Lt = jnp.sum(jnp.where(sel2, vs, 0.0), axis=1, keepdims=True)
    empty = sizes <= 0
    Lt = jnp.where(empty, jnp.float32(jnp.inf), Lt)
    Lt = jnp.where(Q >= 1.0, neg_inf, Lt)

    cnt_gt = jnp.sum((x > Lt).astype(jnp.int32), axis=1, keepdims=True)
    d = sizes - cnt_gt
    ties = jnp.where(x == Lt, col, V)
    it0 = jnp.min(ties, axis=1, keepdims=True)

    def _advance(i, it_cur):
        nxt = jnp.min(jnp.where(ties > it_cur, ties, V), axis=1,
                      keepdims=True)
        return jnp.where(i + 1 < d, nxt, it_cur)

    dmax = jnp.max(d)
    It = jax.lax.fori_loop(0, dmax - 1, _advance, it0)
    It = jnp.where(empty, -1, It)

    mask = (x > Lt) | ((x == Lt) & (col <= It))
    o_ref[...] = mask.astype(jnp.float32)


@jax.jit
def kernel(logits, T, Qhat, msk):
    u = jax.random.uniform(jax.random.key(42), (B,), dtype=jnp.float32)
    t2 = jnp.reshape(T.astype(jnp.float32), (1, 1))
    q2 = jnp.reshape(Qhat.astype(jnp.float32), (1, 1))
    mk = msk[:, :K].astype(jnp.float32)
    u2 = jnp.reshape(u, (B, 1))

    grid = (B // BR,)
    s_mask = pl.pallas_call(
        _body,
        grid=grid,
        in_specs=[
            pl.BlockSpec((BR, V), lambda i: (i, 0)),
            pl.BlockSpec((1, 1), lambda i: (0, 0)),
            pl.BlockSpec((1, 1), lambda i: (0, 0)),
            pl.BlockSpec((1, K), lambda i: (0, 0)),
            pl.BlockSpec((BR, 1), lambda i: (i, 0)),
        ],
        out_specs=pl.BlockSpec((BR, V), lambda i: (i, 0)),
        out_shape=jax.ShapeDtypeStruct((B, V), jnp.float32),
    )(logits, t2, q2, mk, u2)
    return (logits, s_mask)


# speedup vs baseline: 1.0834x; 1.0834x over previous
import functools

import jax
import jax.numpy as jnp
from jax.experimental import pallas as pl

B, V = 128, 100000
K = 10
BR = 16


def _cumsum_cols(a):
    acc = a[:, 0:1]
    cols = [acc]
    for r in range(1, a.shape[1]):
        acc = acc + a[:, r:r + 1]
        cols.append(acc)
    return jnp.concatenate(cols, axis=1)


def _body(x_ref, t_ref, q_ref, mk_ref, u_ref, o_ref):
    x = x_ref[...]
    T = t_ref[0, 0]
    Q = q_ref[0, 0]
    mk = mk_ref[...]
    u = u_ref[...]

    col = jax.lax.broadcasted_iota(jnp.int32, x.shape, 1)
    neg_inf = jnp.float32(-jnp.inf)

    work = x
    vals = []
    for r in range(K):
        m = jnp.max(work, axis=1, keepdims=True)
        vals.append(m)
        if r + 1 < K:
            first = jnp.argmax(work, axis=1).astype(jnp.int32)[:, None]
            work = jnp.where(col == first, neg_inf, work)

    vs = jnp.concatenate(vals, axis=1)

    M = vs[:, 0:1]
    tope = jnp.exp((vs - M) / T)
    Z = jnp.sum(jnp.exp((x - M) / T), axis=1, keepdims=True)
    p = tope / Z

    ordered = p + mk
    cums = _cumsum_cols(p) + _cumsum_cols(mk)

    sizes_base = 1 + jnp.sum((cums <= Q).astype(jnp.int32), axis=1,
                             keepdims=True)
    rk = jax.lax.broadcasted_iota(jnp.int32, (BR, K), 1)
    sel = rk == (sizes_base - 1)
    ord_at = jnp.sum(jnp.where(sel, ordered, 0.0), axis=1, keepdims=True)
    cum_at = jnp.sum(jnp.where(sel, cums, 0.0), axis=1, keepdims=True)
    vprob = (Q - (cum_at - ord_at)) / ord_at
    sizes = sizes_base - (u >= vprob).astype(jnp.int32)

    sel2 = rk == (sizes - 1)
    Lt = jnp.sum(jnp.where(sel2, vs, 0.0), axis=1, keepdims=True)
    empty = sizes <= 0
    Lt = jnp.where(empty, jnp.float32(jnp.inf), Lt)
    Lt = jnp.where(Q >= 1.0, neg_inf, Lt)

    cnt_gt = jnp.sum((x > Lt).astype(jnp.int32), axis=1, keepdims=True)
    d = sizes - cnt_gt
    ties = jnp.where(x == Lt, col, V)
    it0 = jnp.min(ties, axis=1, keepdims=True)

    def _advance(i, it_cur):
        nxt = jnp.min(jnp.where(ties > it_cur, ties, V), axis=1,
                      keepdims=True)
        return jnp.where(i + 1 < d, nxt, it_cur)

    dmax = jnp.max(d)
    It = jax.lax.fori_loop(0, dmax - 1, _advance, it0)
    It = jnp.where(empty, -1, It)

    mask = (x > Lt) | ((x == Lt) & (col <= It))
    o_ref[...] = mask.astype(jnp.float32)


@jax.jit
def kernel(logits, T, Qhat, msk):
    u = jax.random.uniform(jax.random.key(42), (B,), dtype=jnp.float32)
    t2 = jnp.reshape(T.astype(jnp.float32), (1, 1))
    q2 = jnp.reshape(Qhat.astype(jnp.float32), (1, 1))
    mk = msk[:, :K].astype(jnp.float32)
    u2 = jnp.reshape(u, (B, 1))

    grid = (B // BR,)
    s_mask = pl.pallas_call(
        _body,
        grid=grid,
        in_specs=[
            pl.BlockSpec((BR, V), lambda i: (i, 0)),
            pl.BlockSpec((1, 1), lambda i: (0, 0)),
            pl.BlockSpec((1, 1), lambda i: (0, 0)),
            pl.BlockSpec((1, K), lambda i: (0, 0)),
            pl.BlockSpec((BR, 1), lambda i: (i, 0)),
        ],
        out_specs=pl.BlockSpec((BR, V), lambda i: (i, 0)),
        out_shape=jax.ShapeDtypeStruct((B, V), jnp.float32),
    )(logits, t2, q2, mk, u2)
    return (logits, s_mask)


# speedup vs baseline: 1.0850x; 1.0015x over previous
import functools

import jax
import jax.numpy as jnp
from jax.experimental import pallas as pl
from jax.experimental.pallas import tpu as pltpu

B, V = 128, 100000
K = 10
BR = 16


def _cumsum_cols(a):
    acc = a[:, 0:1]
    cols = [acc]
    for r in range(1, a.shape[1]):
        acc = acc + a[:, r:r + 1]
        cols.append(acc)
    return jnp.concatenate(cols, axis=1)


def _body(x_ref, t_ref, q_ref, mk_ref, u_ref, o_ref):
    x = x_ref[...]
    T = t_ref[0, 0]
    Q = q_ref[0, 0]
    mk = mk_ref[...]
    u = u_ref[...]

    col = jax.lax.broadcasted_iota(jnp.int32, x.shape, 1)
    neg_inf = jnp.float32(-jnp.inf)

    work = x
    vals = []
    for r in range(K):
        m = jnp.max(work, axis=1, keepdims=True)
        vals.append(m)
        if r + 1 < K:
            first = jnp.argmax(work, axis=1).astype(jnp.int32)[:, None]
            work = jnp.where(col == first, neg_inf, work)

    vs = jnp.concatenate(vals, axis=1)

    M = vs[:, 0:1]
    tope = jnp.exp((vs - M) / T)
    Z = jnp.sum(jnp.exp((x - M) / T), axis=1, keepdims=True)
    p = tope / Z

    ordered = p + mk
    cums = _cumsum_cols(p) + _cumsum_cols(mk)

    sizes_base = 1 + jnp.sum((cums <= Q).astype(jnp.int32), axis=1,
                             keepdims=True)
    rk = jax.lax.broadcasted_iota(jnp.int32, (BR, K), 1)
    sel = rk == (sizes_base - 1)
    ord_at = jnp.sum(jnp.where(sel, ordered, 0.0), axis=1, keepdims=True)
    cum_at = jnp.sum(jnp.where(sel, cums, 0.0), axis=1, keepdims=True)
    vprob = (Q - (cum_at - ord_at)) / ord_at
    sizes = sizes_base - (u >= vprob).astype(jnp.int32)

    sel2 = rk == (sizes - 1)
    Lt = jnp.sum(jnp.where(sel2, vs, 0.0), axis=1, keepdims=True)
    empty = sizes <= 0
    Lt = jnp.where(empty, jnp.float32(jnp.inf), Lt)
    Lt = jnp.where(Q >= 1.0, neg_inf, Lt)

    cnt_gt = jnp.sum((x > Lt).astype(jnp.int32), axis=1, keepdims=True)
    d = sizes - cnt_gt
    ties = jnp.where(x == Lt, col, V)
    it0 = jnp.min(ties, axis=1, keepdims=True)

    def _advance(i, it_cur):
        nxt = jnp.min(jnp.where(ties > it_cur, ties, V), axis=1,
                      keepdims=True)
        return jnp.where(i + 1 < d, nxt, it_cur)

    dmax = jnp.max(d)
    It = jax.lax.fori_loop(0, dmax - 1, _advance, it0)
    It = jnp.where(empty, -1, It)

    mask = (x > Lt) | ((x == Lt) & (col <= It))
    o_ref[...] = mask.astype(jnp.float32)


@jax.jit
def kernel(logits, T, Qhat, msk):
    u = jax.random.uniform(jax.random.key(42), (B,), dtype=jnp.float32)
    t2 = jnp.reshape(T.astype(jnp.float32), (1, 1))
    q2 = jnp.reshape(Qhat.astype(jnp.float32), (1, 1))
    mk = msk[:, :K].astype(jnp.float32)
    u2 = jnp.reshape(u, (B, 1))

    grid = (B // BR,)
    s_mask = pl.pallas_call(
        _body,
        grid=grid,
        in_specs=[
            pl.BlockSpec((BR, V), lambda i: (i, 0)),
            pl.BlockSpec((1, 1), lambda i: (0, 0)),
            pl.BlockSpec((1, 1), lambda i: (0, 0)),
            pl.BlockSpec((1, K), lambda i: (0, 0)),
            pl.BlockSpec((BR, 1), lambda i: (i, 0)),
        ],
        out_specs=pl.BlockSpec((BR, V), lambda i: (i, 0)),
        out_shape=jax.ShapeDtypeStruct((B, V), jnp.float32),
        compiler_params=pltpu.CompilerParams(
            dimension_semantics=("parallel",)),
    )(logits, t2, q2, mk, u2)
    return (logits, s_mask)


# speedup vs baseline: 1.0850x; 1.0000x over previous
import jax
import jax.numpy as jnp
from jax.experimental import pallas as pl
from jax.experimental.pallas import tpu as pltpu

B, V = 128, 100000
K = 10
BR = 16


def _cumsum_cols(a):
    acc = a[:, 0:1]
    cols = [acc]
    for r in range(1, a.shape[1]):
        acc = acc + a[:, r:r + 1]
        cols.append(acc)
    return jnp.concatenate(cols, axis=1)


def _body(x_ref, t_ref, q_ref, mk_ref, u_ref, o_ref):
    x = x_ref[...]
    T = t_ref[0, 0]
    Q = q_ref[0, 0]
    mk = mk_ref[...]
    u = u_ref[...]

    col = jax.lax.broadcasted_iota(jnp.int32, x.shape, 1)
    neg_inf = jnp.float32(-jnp.inf)

    work = x
    vals = []
    for r in range(K):
        m = jnp.max(work, axis=1, keepdims=True)
        vals.append(m)
        if r + 1 < K:
            first = jnp.argmax(work, axis=1).astype(jnp.int32)[:, None]
            work = jnp.where(col == first, neg_inf, work)

    vs = jnp.concatenate(vals, axis=1)

    M = vs[:, 0:1]
    tope = jnp.exp((vs - M) / T)
    Z = jnp.sum(jnp.exp((x - M) / T), axis=1, keepdims=True)
    p = tope / Z

    ordered = p + mk
    cums = _cumsum_cols(p) + _cumsum_cols(mk)

    sizes_base = 1 + jnp.sum((cums <= Q).astype(jnp.int32), axis=1,
                             keepdims=True)
    rk = jax.lax.broadcasted_iota(jnp.int32, (BR, K), 1)
    sel = rk == (sizes_base - 1)
    ord_at = jnp.sum(jnp.where(sel, ordered, 0.0), axis=1, keepdims=True)
    cum_at = jnp.sum(jnp.where(sel, cums, 0.0), axis=1, keepdims=True)
    vprob = (Q - (cum_at - ord_at)) / ord_at
    sizes = sizes_base - (u >= vprob).astype(jnp.int32)

    sel2 = rk == (sizes - 1)
    Lt = jnp.sum(jnp.where(sel2, vs, 0.0), axis=1, keepdims=True)
    empty = sizes <= 0
    Lt = jnp.where(empty, jnp.float32(jnp.inf), Lt)
    Lt = jnp.where(Q >= 1.0, neg_inf, Lt)

    cnt_gt = jnp.sum((x > Lt).astype(jnp.int32), axis=1, keepdims=True)
    d = sizes - cnt_gt
    ties = jnp.where(x == Lt, col, V)
    it0 = jnp.min(ties, axis=1, keepdims=True)

    def _advance(i, it_cur):
        nxt = jnp.min(jnp.where(ties > it_cur, ties, V), axis=1,
                      keepdims=True)
        return jnp.where(i + 1 < d, nxt, it_cur)

    dmax = jnp.max(d)
    It = jax.lax.fori_loop(0, dmax - 1, _advance, it0)
    It = jnp.where(empty, -1, It)

    mask = (x > Lt) | ((x == Lt) & (col <= It))
    o_ref[...] = mask.astype(jnp.float32)


@jax.jit
def kernel(logits, T, Qhat, msk):
    u = jax.random.uniform(jax.random.key(42), (B,), dtype=jnp.float32)
    t2 = jnp.reshape(T.astype(jnp.float32), (1, 1))
    q2 = jnp.reshape(Qhat.astype(jnp.float32), (1, 1))
    mk = msk[:, :K].astype(jnp.float32)
    u2 = jnp.reshape(u, (B, 1))

    grid = (B // BR,)
    s_mask = pl.pallas_call(
        _body,
        grid=grid,
        in_specs=[
            pl.BlockSpec((BR, V), lambda i: (i, 0)),
            pl.BlockSpec((1, 1), lambda i: (0, 0)),
            pl.BlockSpec((1, 1), lambda i: (0, 0)),
            pl.BlockSpec((1, K), lambda i: (0, 0)),
            pl.BlockSpec((BR, 1), lambda i: (i, 0)),
        ],
        out_specs=pl.BlockSpec((BR, V), lambda i: (i, 0)),
        out_shape=jax.ShapeDtypeStruct((B, V), jnp.float32),
        compiler_params=pltpu.CompilerParams(
            dimension_semantics=("parallel",)),
    )(logits, t2, q2, mk, u2)
    return (logits, s_mask)
